# Initial kernel scaffold; baseline (speedup 1.0000x reference)
import jax
import jax.numpy as jnp
from jax.experimental import pallas as pl


def kernel(cls_logits, cls_targets, box_regression, reg_targets, reg_anchors):
    raise NotImplementedError("write your pallas kernel here")


# speedup vs baseline: 2.0628x; 2.0628x over previous
import functools

import jax
import jax.numpy as jnp
from jax import lax
from jax.experimental import pallas as pl
from jax.experimental.pallas import tpu as pltpu
from jax.experimental.pallas import tpu_sc as plsc

_SC_WORKERS = 32
_CORR_BLOCK = 2048
_DENSE_BLOCK_ROWS = 1000


def _sc_gather_flat(table_flat, idx):
    npad = idx.shape[0]
    rw = npad // _SC_WORKERS
    mesh = plsc.VectorSubcoreMesh(core_axis_name="c", subcore_axis_name="s")

    @functools.partial(
        pl.kernel,
        out_type=jax.ShapeDtypeStruct((npad,), jnp.float32),
        mesh=mesh,
        scratch_types=[
            pltpu.VMEM((rw,), jnp.int32),
            pltpu.VMEM((rw,), jnp.float32),
            pltpu.SemaphoreType.DMA,
        ],
    )
    def k(table_hbm, idx_hbm, out_hbm, idx_v, val_v, sem):
        wid = lax.axis_index("s") * 2 + lax.axis_index("c")
        base = wid * rw
        pltpu.sync_copy(idx_hbm.at[pl.ds(base, rw)], idx_v)
        pltpu.async_copy(table_hbm.at[idx_v], val_v, sem).wait()
        pltpu.sync_copy(val_v, out_hbm.at[pl.ds(base, rw)])

    return k(table_flat, idx)


def _dense_body(x_ref, o_ref):
    i = pl.program_id(0)
    x = x_ref[...]
    ax = jnp.abs(x)
    e = jnp.exp(-ax)
    l1p = jnp.log1p(e)
    r = 1.0 / (1.0 + e)
    sig = jnp.where(x >= 0.0, r, e * r)
    f = (jnp.maximum(x, 0.0) + l1p) * sig * sig

    @pl.when(i == 0)
    def _():
        o_ref[...] = jnp.zeros((1, 1), jnp.float32)

    o_ref[...] += jnp.sum(f).reshape(1, 1)


def _dense_sum(x2):
    nrows = x2.shape[0]
    grid = nrows // _DENSE_BLOCK_ROWS
    return pl.pallas_call(
        _dense_body,
        grid=(grid,),
        in_specs=[pl.BlockSpec((_DENSE_BLOCK_ROWS, 128), lambda i: (i, 0))],
        out_specs=pl.BlockSpec((1, 1), lambda i: (0, 0)),
        out_shape=jax.ShapeDtypeStruct((1, 1), jnp.float32),
    )(x2)


def _corr_body(xp_ref, t_ref, br_ref, rt_ref, an_ref, o_ref):
    i = pl.program_id(0)
    xp = xp_ref[...].reshape(1, _CORR_BLOCK)
    t = t_ref[...].reshape(1, _CORR_BLOCK)
    an = an_ref[...]
    bp = an - br_ref[...]
    bt = an - rt_ref[...]
    lt = jnp.maximum(bp[0:2], bt[0:2])
    rb = jnp.minimum(bp[2:4], bt[2:4])
    wh = jnp.maximum(rb - lt, 0.0)
    ov = wh[0:1] * wh[1:2]
    a1 = (bp[2:3] - bp[0:1]) * (bp[3:4] - bp[1:2])
    a2 = (bt[2:3] - bt[0:1]) * (bt[3:4] - bt[1:2])
    union = a1 + a2 - ov
    iou = ov / jnp.maximum(union, 1e-6)

    pos = t > 0
    s = jnp.where(pos, iou, 0.0)
    ax = jnp.abs(xp)
    e = jnp.exp(-ax)
    l1p = jnp.log1p(e)
    r = 1.0 / (1.0 + e)
    sig = jnp.where(xp >= 0.0, r, e * r)
    relu = jnp.maximum(xp, 0.0)
    d = s - sig
    pos_loss = (relu - xp * s + l1p) * (d * d)
    fxp = (relu + l1p) * sig * sig
    corr = jnp.where(pos, pos_loss - fxp, 0.0)

    @pl.when(i == 0)
    def _():
        o_ref[...] = jnp.zeros((1, 1), jnp.float32)

    o_ref[...] += jnp.sum(corr).reshape(1, 1)


def _corr_sum(xp3, t3, br_t, rt_t, an_t):
    grid = xp3.shape[0]
    box_spec = pl.BlockSpec((4, _CORR_BLOCK), lambda i: (0, i))
    row_spec = pl.BlockSpec((1, 1, _CORR_BLOCK), lambda i: (i, 0, 0))
    return pl.pallas_call(
        _corr_body,
        grid=(grid,),
        in_specs=[row_spec, row_spec, box_spec, box_spec, box_spec],
        out_specs=pl.BlockSpec((1, 1), lambda i: (0, 0)),
        out_shape=jax.ShapeDtypeStruct((1, 1), jnp.float32),
    )(xp3, t3, br_t, rt_t, an_t)


def kernel(cls_logits, cls_targets, box_regression, reg_targets, reg_anchors):
    n, c = cls_logits.shape
    npad = ((n + _CORR_BLOCK - 1) // _CORR_BLOCK) * _CORR_BLOCK

    label = jnp.clip(cls_targets - 1, 0, c - 1)
    flat_idx = jnp.arange(n, dtype=jnp.int32) * c + label
    flat_idx = jnp.pad(flat_idx, (0, npad - n))
    tpadded = jnp.pad(cls_targets, (0, npad - n))

    pad2 = ((0, npad - n), (0, 0))
    br_t = jnp.pad(box_regression, pad2).T
    rt_t = jnp.pad(reg_targets, pad2).T
    an_t = jnp.pad(reg_anchors, pad2).T

    xp = _sc_gather_flat(cls_logits.reshape(n * c), flat_idx)

    dense = _dense_sum(cls_logits.reshape((n * c) // 128, 128))

    nb = npad // _CORR_BLOCK
    corr = _corr_sum(
        xp.reshape(nb, 1, _CORR_BLOCK),
        tpadded.reshape(nb, 1, _CORR_BLOCK),
        br_t,
        rt_t,
        an_t,
    )
    return dense[0, 0] + corr[0, 0]
